# baseline (speedup 1.0000x reference)
import functools

import jax
import jax.numpy as jnp
from jax import lax
from jax.experimental import pallas as pl
from jax.experimental.pallas import tpu as pltpu
from jax.experimental.pallas import tpu_sc as plsc

B = 16384
D = 32
H1 = 256
H2 = 64

_info = plsc.get_sparse_core_info()
_NC, _NS = _info.num_cores, _info.num_subcores
_NW = _NC * _NS
_BPW = B // _NW


def _gather_body(uid_hbm, mid_hbm, utab_hbm, ctab_hbm, uout_hbm, cout_hbm,
                 uidx_v, midx_v, urow_v, crow_v, sem):
    wid = lax.axis_index("s") * _NC + lax.axis_index("c")
    base = wid * _BPW
    pltpu.sync_copy(uid_hbm.at[pl.ds(base, _BPW)], uidx_v)
    pltpu.sync_copy(mid_hbm.at[pl.ds(base, _BPW)], midx_v)
    cp_u = pltpu.async_copy(utab_hbm.at[uidx_v], urow_v, sem)
    cp_c = pltpu.async_copy(ctab_hbm.at[midx_v], crow_v, sem)
    cp_u.wait()
    cp_c.wait()
    pltpu.sync_copy(urow_v, uout_hbm.at[pl.ds(base, _BPW)])
    pltpu.sync_copy(crow_v, cout_hbm.at[pl.ds(base, _BPW)])


_gather = functools.partial(
    pl.kernel,
    mesh=plsc.VectorSubcoreMesh(core_axis_name="c", subcore_axis_name="s"),
    out_type=[
        jax.ShapeDtypeStruct((B, D), jnp.float32),
        jax.ShapeDtypeStruct((B, D), jnp.float32),
    ],
    scratch_types=[
        pltpu.VMEM((_BPW,), jnp.int32),
        pltpu.VMEM((_BPW,), jnp.int32),
        pltpu.VMEM((_BPW, D), jnp.float32),
        pltpu.VMEM((_BPW, D), jnp.float32),
        pltpu.SemaphoreType.DMA,
    ],
    compiler_params=pltpu.CompilerParams(use_tc_tiling_on_sc=False),
)(_gather_body)


def _mlp_body(u_ref, c_ref, w1_ref, b1_ref, w2_ref, b2_ref, w3_ref, b3_ref,
              out_ref):
    u = u_ref[...]
    c = c_ref[...]
    h = jnp.dot(u, w1_ref[0:D, :], preferred_element_type=jnp.float32)
    h += jnp.dot(c, w1_ref[D:2 * D, :], preferred_element_type=jnp.float32)
    h = jnp.maximum(h + b1_ref[...], 0.0)
    h = jnp.dot(h, w2_ref[...], preferred_element_type=jnp.float32)
    h = jnp.maximum(h + b2_ref[...], 0.0)
    out_ref[...] = jnp.dot(h, w3_ref[...],
                           preferred_element_type=jnp.float32) + b3_ref[...]


def _mlp(u, c, W1, b1, W2, b2, W3, b3):
    BM = 2048
    grid = (B // BM,)
    return pl.pallas_call(
        _mlp_body,
        grid=grid,
        in_specs=[
            pl.BlockSpec((BM, D), lambda i: (i, 0)),
            pl.BlockSpec((BM, D), lambda i: (i, 0)),
            pl.BlockSpec((2 * D, H1), lambda i: (0, 0)),
            pl.BlockSpec((H1,), lambda i: (0,)),
            pl.BlockSpec((H1, H2), lambda i: (0, 0)),
            pl.BlockSpec((H2,), lambda i: (0,)),
            pl.BlockSpec((H2, 1), lambda i: (0, 0)),
            pl.BlockSpec((1,), lambda i: (0,)),
        ],
        out_specs=pl.BlockSpec((BM, 1), lambda i: (i, 0)),
        out_shape=jax.ShapeDtypeStruct((B, 1), jnp.float32),
    )(u, c, W1, b1, W2, b2, W3, b3)


def kernel(user_id, movie_id, user_table, cand_table, W1, b1, W2, b2, W3, b3):
    u_rows, c_rows = _gather(user_id.astype(jnp.int32),
                             movie_id.astype(jnp.int32),
                             user_table, cand_table)
    return _mlp(u_rows, c_rows, W1, b1, W2, b2, W3, b3)


# speedup vs baseline: 1.3600x; 1.3600x over previous
import functools

import jax
import jax.numpy as jnp
from jax import lax
from jax.experimental import pallas as pl
from jax.experimental.pallas import tpu as pltpu
from jax.experimental.pallas import tpu_sc as plsc

B = 16384
V = 1000000
D = 32
H1 = 256
H2 = 64

_info = plsc.get_sparse_core_info()
_NC, _NS = _info.num_cores, _info.num_subcores
_NW = _NC * _NS
_BPW = B // _NW
_TCHUNK = 64
_L = 16


def _gather_body(uid_hbm, mid_hbm, utab_hbm, ctab_hbm, uout_hbm, cout_hbm,
                 uidx_v, midx_v, uq_v, mq_v, tiles_v,
                 urow_v, crow_v, sem):
    wid = lax.axis_index("s") * _NC + lax.axis_index("c")
    base = wid * _BPW
    pltpu.sync_copy(uid_hbm.at[pl.ds(base, _BPW)], uidx_v)
    pltpu.sync_copy(mid_hbm.at[pl.ds(base, _BPW)], midx_v)

    for k0 in range(0, _BPW, _L):
        sl = pl.ds(k0, _L)
        uq_v[sl] = jnp.bitwise_and(uidx_v[sl], 7)
        mq_v[sl] = jnp.bitwise_and(midx_v[sl], 7)

    lanes = lax.iota(jnp.int32, _L)

    def extract(q_src, row_dst, c0):
        for sub in range(_TCHUNK // _L):
            jb = sub * _L
            q = q_src[pl.ds(c0 + jb, _L)]
            slot = lanes + jb
            for c in range(D):
                cv = jnp.full((_L,), c, jnp.int32)
                vec = plsc.load_gather(tiles_v, [slot, q, cv])
                row_dst[c, pl.ds(c0 + jb, _L)] = vec

    def fetch(idx_v, tab_hbm, c0):
        cps = []
        for sub in range(_TCHUNK // _L):
            w = idx_v[pl.ds(c0 + sub * _L, _L)]
            for j in range(_L):
                r = jnp.sum(jnp.where(lanes == j, w, 0))
                rb = pl.multiple_of(jnp.bitwise_and(r, -8), 8)
                cps.append(pltpu.async_copy(tab_hbm.at[pl.ds(rb, 8)],
                                            tiles_v.at[sub * _L + j], sem))
        for cp in cps:
            cp.wait()

    def chunk_body(i, _):
        c0 = i * _TCHUNK
        fetch(uidx_v, utab_hbm, c0)
        extract(uq_v, urow_v, c0)
        fetch(midx_v, ctab_hbm, c0)
        extract(mq_v, crow_v, c0)
        return ()

    lax.fori_loop(0, _BPW // _TCHUNK, chunk_body, ())

    pltpu.sync_copy(urow_v, uout_hbm.at[:, pl.ds(base, _BPW)])
    pltpu.sync_copy(crow_v, cout_hbm.at[:, pl.ds(base, _BPW)])


_gather = functools.partial(
    pl.kernel,
    mesh=plsc.VectorSubcoreMesh(core_axis_name="c", subcore_axis_name="s"),
    out_type=[
        jax.ShapeDtypeStruct((D, B), jnp.float32),
        jax.ShapeDtypeStruct((D, B), jnp.float32),
    ],
    scratch_types=[
        pltpu.VMEM((_BPW,), jnp.int32),
        pltpu.VMEM((_BPW,), jnp.int32),
        pltpu.VMEM((_BPW,), jnp.int32),
        pltpu.VMEM((_BPW,), jnp.int32),
        pltpu.VMEM((_TCHUNK, 8, D), jnp.float32),
        pltpu.VMEM((D, _BPW), jnp.float32),
        pltpu.VMEM((D, _BPW), jnp.float32),
        pltpu.SemaphoreType.DMA,
    ],
    compiler_params=pltpu.CompilerParams(needs_layout_passes=False),
)(_gather_body)


def _mlp_body(u_ref, c_ref, w1_ref, b1_ref, w2_ref, b2_ref, w3_ref, b3_ref,
              out_ref):
    uT = u_ref[...]
    cT = c_ref[...]
    ct = (((0,), (0,)), ((), ()))
    h = lax.dot_general(w1_ref[0:D, :], uT, ct,
                        preferred_element_type=jnp.float32)
    h += lax.dot_general(w1_ref[D:2 * D, :], cT, ct,
                         preferred_element_type=jnp.float32)
    h = jnp.maximum(h + b1_ref[...], 0.0)
    h = lax.dot_general(w2_ref[...], h, ct,
                        preferred_element_type=jnp.float32)
    h = jnp.maximum(h + b2_ref[...], 0.0)
    out_ref[...] = lax.dot_general(w3_ref[...], h, ct,
                                   preferred_element_type=jnp.float32) \
        + b3_ref[...]


def _mlp(uT, cT, W1, b1, W2, b2, W3, b3):
    BM = 2048
    grid = (B // BM,)
    return pl.pallas_call(
        _mlp_body,
        grid=grid,
        in_specs=[
            pl.BlockSpec((D, BM), lambda i: (0, i)),
            pl.BlockSpec((D, BM), lambda i: (0, i)),
            pl.BlockSpec((2 * D, H1), lambda i: (0, 0)),
            pl.BlockSpec((H1, 1), lambda i: (0, 0)),
            pl.BlockSpec((H1, H2), lambda i: (0, 0)),
            pl.BlockSpec((H2, 1), lambda i: (0, 0)),
            pl.BlockSpec((H2, 1), lambda i: (0, 0)),
            pl.BlockSpec((1, 1), lambda i: (0, 0)),
        ],
        out_specs=pl.BlockSpec((1, BM), lambda i: (0, i)),
        out_shape=jax.ShapeDtypeStruct((1, B), jnp.float32),
    )(uT, cT, W1, b1, W2, b2, W3, b3)


def kernel(user_id, movie_id, user_table, cand_table, W1, b1, W2, b2, W3, b3):
    uT, cT = _gather(user_id.astype(jnp.int32), movie_id.astype(jnp.int32),
                     user_table, cand_table)
    outT = _mlp(uT, cT, W1, b1.reshape(H1, 1), W2, b2.reshape(H2, 1),
                W3, b3.reshape(1, 1))
    return outT.reshape(B, 1)
